# baseline (speedup 1.0000x reference)
import jax
import jax.numpy as jnp
from jax import lax
from jax.experimental import pallas as pl
from jax.experimental.pallas import tpu as pltpu
from jax.experimental.pallas import tpu_sc as plsc

NC = 2
NS = 16
L = 16
NW = NC * NS

B = 16384
D = 64
D2 = 2 * D
BPW = B // NW
PASS = BPW // 2
CHUNK = 128
NCH = PASS // CHUNK
GROUPS = PASS // L


def _mf_body(uidx_hbm, iidx_hbm, utab_hbm, itab_hbm, out_hbm,
             uidx_v, iidx_v, ug_v, ig_v, upair_v, ipair_v, out_v, sem):
    wid = lax.axis_index("s") * NC + lax.axis_index("c")
    base = wid * BPW

    pltpu.sync_copy(uidx_hbm.at[wid], uidx_v)
    pltpu.sync_copy(iidx_hbm.at[wid], iidx_v)

    def shift_body(i, carry):
        ug_v[pl.ds(i * L, L)] = lax.shift_right_logical(
            uidx_v[pl.ds(i * L, L)], 1)
        ig_v[pl.ds(i * L, L)] = lax.shift_right_logical(
            iidx_v[pl.ds(i * L, L)], 1)
        return carry

    lax.fori_loop(0, BPW // L, shift_body, 0)

    lane = lax.broadcasted_iota(jnp.int32, (L,), 0)
    one = jnp.ones((L,), jnp.int32)

    for p in range(2):
        copies = []
        for c in range(NCH):
            off = p * PASS + c * CHUNK
            copies.append(pltpu.async_copy(
                utab_hbm.at[ug_v.at[pl.ds(off, CHUNK)]],
                upair_v.at[pl.ds(c * CHUNK, CHUNK)], sem))
            copies.append(pltpu.async_copy(
                itab_hbm.at[ig_v.at[pl.ds(off, CHUNK)]],
                ipair_v.at[pl.ds(c * CHUNK, CHUNK)], sem))
        for cp in copies:
            cp.wait()

        def group_body(g, carry):
            row0 = g * L
            ll = jnp.zeros((L,), jnp.float32)
            lh = jnp.zeros((L,), jnp.float32)
            hl = jnp.zeros((L,), jnp.float32)
            hh = jnp.zeros((L,), jnp.float32)
            for r in range(L):
                row = row0 + r
                sll = jnp.zeros((L,), jnp.float32)
                slh = jnp.zeros((L,), jnp.float32)
                shl = jnp.zeros((L,), jnp.float32)
                shh = jnp.zeros((L,), jnp.float32)
                for k in range(D // L):
                    ulo = upair_v[row, pl.ds(k * L, L)]
                    uhi = upair_v[row, pl.ds(D + k * L, L)]
                    ilo = ipair_v[row, pl.ds(k * L, L)]
                    ihi = ipair_v[row, pl.ds(D + k * L, L)]
                    sll = sll + ulo * ilo
                    slh = slh + ulo * ihi
                    shl = shl + uhi * ilo
                    shh = shh + uhi * ihi
                sel = lane == r
                ll = jnp.where(sel, jnp.sum(sll), ll)
                lh = jnp.where(sel, jnp.sum(slh), lh)
                hl = jnp.where(sel, jnp.sum(shl), hl)
                hh = jnp.where(sel, jnp.sum(shh), hh)
            boff = p * PASS + row0
            pu = (uidx_v[pl.ds(boff, L)] & one) == one
            pi = (iidx_v[pl.ds(boff, L)] & one) == one
            out_v[pl.ds(boff, L)] = jnp.where(
                pu, jnp.where(pi, hh, hl), jnp.where(pi, lh, ll))
            return carry

        lax.fori_loop(0, GROUPS, group_body, 0)

    pltpu.sync_copy(out_v, out_hbm.at[pl.ds(base, BPW)])


_mf_call = pl.kernel(
    _mf_body,
    out_type=jax.ShapeDtypeStruct((B,), jnp.float32),
    mesh=plsc.VectorSubcoreMesh(core_axis_name="c", subcore_axis_name="s"),
    compiler_params=pltpu.CompilerParams(
        needs_layout_passes=False, use_tc_tiling_on_sc=True),
    scratch_types=[
        pltpu.VMEM((BPW,), jnp.int32),
        pltpu.VMEM((BPW,), jnp.int32),
        pltpu.VMEM((BPW,), jnp.int32),
        pltpu.VMEM((BPW,), jnp.int32),
        pltpu.VMEM((PASS, D2), jnp.float32),
        pltpu.VMEM((PASS, D2), jnp.float32),
        pltpu.VMEM((BPW,), jnp.float32),
        pltpu.SemaphoreType.DMA,
    ],
)


def _pair(table):
    n = table.shape[0] - 1
    return jnp.concatenate([table[0:n:2], table[1:n:2]], axis=1)


@jax.jit
def kernel(user_indices, item_indices, user_table, item_table):
    uidx = user_indices.astype(jnp.int32).reshape(NW, BPW)
    iidx = item_indices.astype(jnp.int32).reshape(NW, BPW)
    return _mf_call(uidx, iidx, _pair(user_table), _pair(item_table))


# speedup vs baseline: 21.7872x; 21.7872x over previous
import jax
import jax.numpy as jnp
from jax import lax
from jax.experimental import pallas as pl
from jax.experimental.pallas import tpu as pltpu
from jax.experimental.pallas import tpu_sc as plsc

NC = 2
NS = 16
L = 16
NW = NC * NS

B = 16384
D = 64
BPW = B // NW
PE = 16
NP = BPW // PE


def _mf_body(uidx_hbm, iidx_hbm, utab_hbm, itab_hbm, out_hbm,
             uidx_v, iidx_v, ua_v, ia_v, ub_v, ib_v, out_v, sema, semb):
    wid = lax.axis_index("s") * NC + lax.axis_index("c")
    base = wid * BPW

    pltpu.sync_copy(uidx_hbm.at[wid], uidx_v)
    pltpu.sync_copy(iidx_hbm.at[wid], iidx_v)

    lane = lax.broadcasted_iota(jnp.int32, (L,), 0)

    def issue(p, ubuf, ibuf, sem):
        vecu = uidx_v[0, pl.ds(p * PE, L)]
        veci = iidx_v[0, pl.ds(p * PE, L)]
        for r in range(L):
            ub = pl.multiple_of((vecu[r] >> 3) * 8, 8)
            ib = pl.multiple_of((veci[r] >> 3) * 8, 8)
            pltpu.make_async_copy(
                utab_hbm.at[pl.ds(ub, 8), :],
                ubuf.at[pl.ds(r * 8, 8), :], sem).start()
            pltpu.make_async_copy(
                itab_hbm.at[pl.ds(ib, 8), :],
                ibuf.at[pl.ds(r * 8, 8), :], sem).start()

    def drain(ubuf, ibuf, sem):
        pltpu.make_async_copy(
            utab_hbm.at[pl.ds(0, PE * 8), :], ubuf, sem).wait()
        pltpu.make_async_copy(
            itab_hbm.at[pl.ds(0, PE * 8), :], ibuf, sem).wait()

    def compute(p, ubuf, ibuf):
        vecu = uidx_v[0, pl.ds(p * PE, L)]
        veci = iidx_v[0, pl.ds(p * PE, L)]
        vec = jnp.zeros((L,), jnp.float32)
        for r in range(L):
            su = r * 8 + (vecu[r] & 7)
            si = r * 8 + (veci[r] & 7)
            acc = ubuf[su, pl.ds(0, L)] * ibuf[si, pl.ds(0, L)]
            for k in range(1, D // L):
                acc = acc + (ubuf[su, pl.ds(k * L, L)]
                             * ibuf[si, pl.ds(k * L, L)])
            vec = jnp.where(lane == r, jnp.sum(acc), vec)
        out_v[pl.ds(p * PE, L)] = vec

    issue(0, ua_v, ia_v, sema)

    def pipe_body(k, carry):
        p = k * 2
        issue(p + 1, ub_v, ib_v, semb)
        drain(ua_v, ia_v, sema)
        compute(p, ua_v, ia_v)

        @pl.when(k < NP // 2 - 1)
        def _():
            issue(p + 2, ua_v, ia_v, sema)

        drain(ub_v, ib_v, semb)
        compute(p + 1, ub_v, ib_v)
        return carry

    lax.fori_loop(0, NP // 2, pipe_body, 0)

    pltpu.sync_copy(out_v, out_hbm.at[pl.ds(base, BPW)])


_mf_call = pl.kernel(
    _mf_body,
    out_type=jax.ShapeDtypeStruct((B,), jnp.float32),
    mesh=plsc.VectorSubcoreMesh(core_axis_name="c", subcore_axis_name="s"),
    compiler_params=pltpu.CompilerParams(
        needs_layout_passes=False, use_tc_tiling_on_sc=True),
    scratch_types=[
        pltpu.VMEM((1, BPW), jnp.int32),
        pltpu.VMEM((1, BPW), jnp.int32),
        pltpu.VMEM((PE * 8, D), jnp.float32),
        pltpu.VMEM((PE * 8, D), jnp.float32),
        pltpu.VMEM((PE * 8, D), jnp.float32),
        pltpu.VMEM((PE * 8, D), jnp.float32),
        pltpu.VMEM((BPW,), jnp.float32),
        pltpu.SemaphoreType.DMA,
        pltpu.SemaphoreType.DMA,
    ],
)


@jax.jit
def kernel(user_indices, item_indices, user_table, item_table):
    uidx = user_indices.astype(jnp.int32).reshape(NW, 1, BPW)
    iidx = item_indices.astype(jnp.int32).reshape(NW, 1, BPW)
    return _mf_call(uidx, iidx, user_table, item_table)
